# baseline (speedup 1.0000x reference)
import functools

import jax
import jax.numpy as jnp
from jax import lax
from jax.experimental import pallas as pl
from jax.experimental.pallas import tpu as pltpu
from jax.experimental.pallas import tpu_sc as plsc

NF = 26
V = 100000
E = 32
B = 16384
H = 128
IN_DIM = NF * E

NC = 2
NS = 16
NW = NC * NS

TCH = 512
NFULL = V // TCH
NCHT = NF * NFULL
VMID = NFULL * TCH
VTAIL = 99968

R = NF * B
RPW = R // NW
DCH = 128
FIRE = 8
BUF = DCH * FIRE
NOUT = RPW // BUF
NIDX = RPW // DCH


def _transpose_chunk(in_v, out_v, nq):
    iot = lax.iota(jnp.int32, 16)
    iot16 = iot + 16

    @plsc.parallel_loop(0, nq, unroll=16)
    def _(q):
        base = jnp.broadcast_to(4 * q, (16,))
        for d in range(4):
            cols = base + d if d else base
            out_v[q, pl.ds(32 * d, 16)] = plsc.load_gather(in_v, [iot, cols])
            out_v[q, pl.ds(32 * d + 16, 16)] = plsc.load_gather(
                in_v, [iot16, cols]
            )


def _make_transpose():
    mesh = plsc.VectorSubcoreMesh(core_axis_name="c", subcore_axis_name="s")

    @functools.partial(
        pl.kernel,
        mesh=mesh,
        out_type=jax.ShapeDtypeStruct((NF * V // 4, 128), jnp.float32),
        compiler_params=pltpu.CompilerParams(
            use_tc_tiling_on_sc=True, needs_layout_passes=False
        ),
        scratch_types=[
            pltpu.VMEM((E, TCH), jnp.float32),
            pltpu.VMEM((TCH // 4, 128), jnp.float32),
        ],
    )
    def transpose_k(tab_hbm, tail_hbm, out_hbm, in_v, out_v):
        wid = lax.axis_index("s") * NC + lax.axis_index("c")

        def full_chunk(k, carry):
            chunk = wid + k * NW
            f = chunk // NFULL
            c = chunk % NFULL
            pltpu.sync_copy(tab_hbm.at[f, :, pl.ds(c * TCH, TCH)], in_v)
            _transpose_chunk(in_v, out_v, TCH // 4)
            qbase = pl.multiple_of((f * V + c * TCH) // 4, 128)
            pltpu.sync_copy(out_v, out_hbm.at[pl.ds(qbase, TCH // 4)])
            return carry

        nloc = (NCHT - wid + NW - 1) // NW
        lax.fori_loop(0, nloc, full_chunk, 0, unroll=False)

        @pl.when(wid < NF)
        def _():
            f = wid
            pltpu.sync_copy(
                tab_hbm.at[f, :, pl.ds(VMID, 128)], in_v.at[:, pl.ds(0, 128)]
            )
            _transpose_chunk(in_v, out_v, 32)
            qmid = pl.multiple_of((f * V + VMID) // 4, 8)
            pltpu.sync_copy(
                out_v.at[pl.ds(0, 32)], out_hbm.at[pl.ds(qmid, 32)]
            )
            pltpu.sync_copy(tail_hbm.at[pl.ds(f * 8, 8)], out_v.at[pl.ds(0, 8)])
            qtail = pl.multiple_of((f * V + VTAIL) // 4, 8)
            pltpu.sync_copy(
                out_v.at[pl.ds(0, 8)], out_hbm.at[pl.ds(qtail, 8)]
            )

    return transpose_k


_transpose = _make_transpose()


def _make_gather():
    mesh = plsc.VectorSubcoreMesh(core_axis_name="c", subcore_axis_name="s")

    @functools.partial(
        pl.kernel,
        mesh=mesh,
        out_type=jax.ShapeDtypeStruct((R, E), jnp.float32),
        compiler_params=pltpu.CompilerParams(use_tc_tiling_on_sc=False),
        scratch_types=[
            pltpu.VMEM((NIDX, DCH), jnp.int32),
            pltpu.VMEM((BUF, E), jnp.float32),
            pltpu.SemaphoreType.DMA,
        ],
    )
    def gather_k(tab_hbm, idx_hbm, out_hbm, idx_v, buf, sem):
        wid = lax.axis_index("s") * NC + lax.axis_index("c")
        base = wid * RPW
        pltpu.sync_copy(idx_hbm.at[wid], idx_v)

        def body(k, carry):
            descs = []
            for j in range(FIRE):
                c = wid * NIDX + k * FIRE + j
                f = c // (B // DCH)
                descs.append(
                    pltpu.async_copy(
                        tab_hbm.at[f].at[idx_v.at[k * FIRE + j]],
                        buf.at[pl.ds(j * DCH, DCH)],
                        sem,
                    )
                )
            for d in descs:
                d.wait()
            pltpu.sync_copy(buf, out_hbm.at[pl.ds(base + k * BUF, BUF)])
            return carry

        lax.fori_loop(0, NOUT, body, 0)

    return gather_k


_gather = _make_gather()


def _mm_body(emb_ref, w_ref, b_ref, o_ref):
    acc = jnp.dot(emb_ref[0], w_ref[0], preferred_element_type=jnp.float32)
    for f in range(1, NF):
        acc += jnp.dot(emb_ref[f], w_ref[f], preferred_element_type=jnp.float32)
    o_ref[...] = jnp.tanh(acc + b_ref[...])


BB = 1024

_matmul = pl.pallas_call(
    _mm_body,
    grid=(B // BB,),
    in_specs=[
        pl.BlockSpec((NF, BB // 4, 128), lambda i: (0, i, 0)),
        pl.BlockSpec((NF, 128, 4 * H), lambda i: (0, 0, 0)),
        pl.BlockSpec((1, 4 * H), lambda i: (0, 0)),
    ],
    out_specs=pl.BlockSpec((BB // 4, 4 * H), lambda i: (i, 0)),
    out_shape=jax.ShapeDtypeStruct((B // 4, 4 * H), jnp.float32),
)


def kernel(x, tables, W, b):
    tail_packed = tables[:, VTAIL:, :].reshape(NF, 8, 4 * E).reshape(NF * 8, 128)
    tab_lin = _transpose(jnp.transpose(tables, (0, 2, 1)), tail_packed)
    tab3 = tab_lin.reshape(NF, V, E)
    g_idx = x.reshape(NW, NIDX, DCH)
    emb_q = _gather(tab3, g_idx).reshape(NF, B // 4, 128)
    w_r = W.reshape(NF, E, H)
    w_big = jnp.einsum(
        "dD,feh->fdeDh", jnp.eye(4, dtype=W.dtype), w_r
    ).reshape(NF, 128, 4 * H)
    b_big = jnp.tile(b, 4).reshape(1, 4 * H)
    out_q = _matmul(emb_q, w_big, b_big)
    return out_q.reshape(B, H)


# speedup vs baseline: 1.3327x; 1.3327x over previous
import functools

import jax
import jax.numpy as jnp
from jax import lax
from jax.experimental import pallas as pl
from jax.experimental.pallas import tpu as pltpu
from jax.experimental.pallas import tpu_sc as plsc

NF = 26
V = 100000
E = 32
B = 16384
H = 128
IN_DIM = NF * E

NC = 2
NS = 16
NW = NC * NS

R = NF * B
RPW = R // NW
DCH = 128
FIRE = 8
BUF = DCH * FIRE
NOUT = RPW // BUF
NIDX = RPW // DCH


def _make_gather():
    mesh = plsc.VectorSubcoreMesh(core_axis_name="c", subcore_axis_name="s")

    @functools.partial(
        pl.kernel,
        mesh=mesh,
        out_type=jax.ShapeDtypeStruct((R, E), jnp.float32),
        compiler_params=pltpu.CompilerParams(use_tc_tiling_on_sc=False),
        scratch_types=[
            pltpu.VMEM((NIDX, DCH), jnp.int32),
            pltpu.VMEM((BUF, E), jnp.float32),
            pltpu.SemaphoreType.DMA,
        ],
    )
    def gather_k(tab_hbm, idx_hbm, out_hbm, idx_v, buf, sem):
        wid = lax.axis_index("s") * NC + lax.axis_index("c")
        base = wid * RPW
        pltpu.sync_copy(idx_hbm.at[wid], idx_v)

        def body(k, carry):
            descs = []
            for j in range(FIRE):
                descs.append(
                    pltpu.async_copy(
                        tab_hbm.at[idx_v.at[k * FIRE + j]],
                        buf.at[pl.ds(j * DCH, DCH)],
                        sem,
                    )
                )
            for d in descs:
                d.wait()
            pltpu.sync_copy(buf, out_hbm.at[pl.ds(base + k * BUF, BUF)])
            return carry

        lax.fori_loop(0, NOUT, body, 0)

    return gather_k


_gather = _make_gather()


def _mm_body(cat_ref, w_ref, b_ref, o_ref):
    acc = jnp.dot(cat_ref[...], w_ref[...], preferred_element_type=jnp.float32)
    o_ref[...] = jnp.tanh(acc + b_ref[...])


BB = 1024

_matmul = pl.pallas_call(
    _mm_body,
    grid=(B // BB,),
    in_specs=[
        pl.BlockSpec((BB, IN_DIM), lambda i: (i, 0)),
        pl.BlockSpec((IN_DIM, H), lambda i: (0, 0)),
        pl.BlockSpec((1, H), lambda i: (0, 0)),
    ],
    out_specs=pl.BlockSpec((BB, H), lambda i: (i, 0)),
    out_shape=jax.ShapeDtypeStruct((B, H), jnp.float32),
)


def kernel(x, tables, W, b):
    offs = (jnp.arange(NF, dtype=jnp.int32) * V)[:, None]
    g_idx = (x + offs).T.reshape(NW, NIDX, DCH)
    tab = tables.reshape(NF * V, E)
    cat = _gather(tab, g_idx).reshape(B, IN_DIM)
    return _matmul(cat, W, b.reshape(1, H))
